# baseline (speedup 1.0000x reference)
import functools

import jax
import jax.numpy as jnp
from jax import lax
from jax.experimental import pallas as pl
from jax.experimental.pallas import tpu as pltpu
from jax.experimental.pallas import tpu_sc as plsc

_NC = 2
_NS = 16
_NW = _NC * _NS
_L = 16

_CHUNK = 128
_QS = 8
_SUPER = _CHUNK * _QS


def _round_up(v, m):
    return (v + m - 1) // m * m


def _spmm_sc(h, meta, npad):
    nchunks = meta.shape[0]
    per_w_chunks = nchunks // _NW
    nsuper = per_w_chunks // _QS
    rpt = npad // _NS

    mesh = plsc.VectorSubcoreMesh(core_axis_name="c", subcore_axis_name="s")

    @functools.partial(
        pl.kernel,
        out_type=jax.ShapeDtypeStruct((_NC, npad, _L), jnp.float32),
        mesh=mesh,
        scratch_types=[
            pltpu.VMEM((2, _QS, 3, _CHUNK), jnp.int32),
            pltpu.VMEM((2, _QS, _CHUNK, _L), jnp.float32),
            pltpu.VMEM((rpt, _L), jnp.float32),
            pltpu.VMEM_SHARED((npad, _L), jnp.float32),
            pltpu.SemaphoreType.DMA,
            pltpu.SemaphoreType.DMA,
            pltpu.SemaphoreType.DMA,
            pltpu.SemaphoreType.DMA,
        ],
        compiler_params=pltpu.CompilerParams(
            use_tc_tiling_on_sc=False, needs_layout_passes=False),
    )
    def spmm(h_hbm, meta_hbm, out_hbm,
             metab, rowsb, stage, acc, g0, g1, s0, s1):
        c = lax.axis_index("c")
        s = lax.axis_index("s")
        wid = c * _NS + s
        chunk0 = wid * per_w_chunks
        gsem = (g0, g1)
        ssem = (s0, s1)

        zero = jnp.zeros((_L,), jnp.float32)

        @pl.loop(0, rpt)
        def _(i):
            stage[i, :] = zero

        pltpu.sync_copy(stage, acc.at[pl.ds(s * rpt, rpt)])
        plsc.subcore_barrier()

        gds = [None, None]
        sds = [None, None]

        def stage_in(slot, t):
            off = chunk0 + t * _QS
            pltpu.sync_copy(meta_hbm.at[pl.ds(off, _QS)], metab.at[slot])
            gds[slot] = [
                pltpu.async_copy(h_hbm.at[metab.at[slot, q, 0]],
                                 rowsb.at[slot, q], gsem[slot])
                for q in range(_QS)
            ]

        def scale(slot):
            iota = lax.iota(jnp.int32, _L)
            sl = jnp.full((_L,), slot, jnp.int32)
            gpc = _CHUNK // _L

            @pl.loop(0, _QS * gpc)
            def _(g):
                q = g // gpc
                r = g - q * gpc
                wvec = plsc.bitcast(
                    metab[slot, q, 2, pl.ds(r * _L, _L)], jnp.float32)
                qq = jnp.full((_L,), q, jnp.int32)
                eidx = r * _L + iota
                for j in range(_L):
                    jj = jnp.full((_L,), j, jnp.int32)
                    col = plsc.load_gather(rowsb, [sl, qq, eidx, jj])
                    plsc.store_scatter(rowsb, [sl, qq, eidx, jj],
                                       col * wvec)

        def fire_scatter(slot):
            sds[slot] = [
                pltpu.async_copy(rowsb.at[slot, q],
                                 acc.at[metab.at[slot, q, 1]],
                                 ssem[slot], add=True)
                for q in range(_QS)
            ]

        stage_in(0, 0)
        for t in range(nsuper):
            slot = t & 1
            if t + 1 < nsuper:
                if sds[slot ^ 1] is not None:
                    for d in sds[slot ^ 1]:
                        d.wait()
                    sds[slot ^ 1] = None
                stage_in(slot ^ 1, t + 1)
            for d in gds[slot]:
                d.wait()
            gds[slot] = None
            scale(slot)
            fire_scatter(slot)
        for slot in (0, 1):
            if sds[slot] is not None:
                for d in sds[slot]:
                    d.wait()
                sds[slot] = None

        plsc.subcore_barrier()
        pltpu.sync_copy(acc.at[pl.ds(s * rpt, rpt)], stage)
        pltpu.sync_copy(stage, out_hbm.at[c, pl.ds(s * rpt, rpt)])

    return spmm(h, meta)


def _mm_body(a_ref, b_ref, o_ref):
    o_ref[...] = jnp.dot(a_ref[...], b_ref[...],
                         preferred_element_type=jnp.float32)


def _dense_mm(a, b):
    return pl.pallas_call(
        _mm_body,
        out_shape=jax.ShapeDtypeStruct((a.shape[0], b.shape[1]), jnp.float32),
    )(a, b)


def _layer2_body(p_ref, b1_ref, w2_ref, o_ref):
    hid = jnp.maximum(p_ref[0] + p_ref[1] + b1_ref[...], 0.0)
    o_ref[...] = jnp.dot(hid, w2_ref[...], preferred_element_type=jnp.float32)


def _layer2(partials, b1row, w2pad):
    npad = partials.shape[1]
    return pl.pallas_call(
        _layer2_body,
        out_shape=jax.ShapeDtypeStruct((npad, _L), jnp.float32),
    )(partials, b1row, w2pad)


def _final_body(p_ref, b2_ref, o_ref):
    o_ref[...] = p_ref[0] + p_ref[1] + b2_ref[...]


def _final(partials, b2row):
    npad = partials.shape[1]
    return pl.pallas_call(
        _final_body,
        out_shape=jax.ShapeDtypeStruct((npad, _L), jnp.float32),
    )(partials, b2row)


def kernel(x, edge_index, edge_weight, W1, b1, W2, b2):
    n, d = x.shape
    h1w = W1.shape[1]
    h2w = W2.shape[1]
    e = edge_index.shape[1]

    npad = _round_up(n, _NS * 8)

    per_w = _round_up(_round_up(e, _NW) // _NW, _SUPER)
    epad = _NW * per_w
    pad = epad - e
    src2d = jnp.pad(edge_index[0].astype(jnp.int32), (0, pad)).reshape(
        -1, _CHUNK)
    dst2d = jnp.pad(edge_index[1].astype(jnp.int32), (0, pad)).reshape(
        -1, _CHUNK)
    w2d = lax.bitcast_convert_type(
        jnp.pad(edge_weight.astype(jnp.float32), (0, pad)), jnp.int32
    ).reshape(-1, _CHUNK)
    meta = jnp.stack([src2d, dst2d, w2d], axis=1)

    w1pad = jnp.pad(W1, ((0, 0), (0, _L - h1w)))
    b1row = jnp.pad(b1, (0, _L - h1w)).reshape(1, _L)
    w2pad = jnp.pad(W2, ((0, _L - h1w), (0, _L - h2w)))
    b2row = jnp.pad(b2, (0, _L - h2w)).reshape(1, _L)

    s1 = _dense_mm(x, w1pad)
    p1 = _spmm_sc(s1, meta, npad)
    s2 = _layer2(p1, b1row, w2pad)
    p2 = _spmm_sc(s2, meta, npad)
    outp = _final(p2, b2row)
    return outp[:n, :h2w]


# speedup vs baseline: 1.4036x; 1.4036x over previous
import functools

import jax
import jax.numpy as jnp
from jax import lax
from jax.experimental import pallas as pl
from jax.experimental.pallas import tpu as pltpu
from jax.experimental.pallas import tpu_sc as plsc

_NC = 2
_NS = 16
_NW = _NC * _NS
_L = 16

_CHUNK = 128
_QS = 8
_SUPER = _CHUNK * _QS


def _round_up(v, m):
    return (v + m - 1) // m * m


def _spmm_sc(h, meta, npad):
    nchunks = meta.shape[0]
    per_w_chunks = nchunks // _NW
    nsuper = per_w_chunks // _QS
    rpt = npad // _NS

    mesh = plsc.VectorSubcoreMesh(core_axis_name="c", subcore_axis_name="s")

    @functools.partial(
        pl.kernel,
        out_type=jax.ShapeDtypeStruct((_NC, npad, _L), jnp.float32),
        mesh=mesh,
        scratch_types=[
            pltpu.VMEM((2, _QS, 3, _CHUNK), jnp.int32),
            pltpu.VMEM((2, _QS, _CHUNK, _L), jnp.float32),
            pltpu.VMEM((rpt, _L), jnp.float32),
            pltpu.VMEM_SHARED((npad, _L), jnp.float32),
            pltpu.SemaphoreType.DMA,
            pltpu.SemaphoreType.DMA,
            pltpu.SemaphoreType.DMA,
            pltpu.SemaphoreType.DMA,
        ],
        compiler_params=pltpu.CompilerParams(
            use_tc_tiling_on_sc=False, needs_layout_passes=False),
    )
    def spmm(h_hbm, meta_hbm, out_hbm,
             metab, rowsb, stage, acc, g0, g1, s0, s1):
        c = lax.axis_index("c")
        s = lax.axis_index("s")
        wid = c * _NS + s
        chunk0 = wid * per_w_chunks
        gsem = (g0, g1)
        ssem = (s0, s1)

        zero = jnp.zeros((_L,), jnp.float32)

        @pl.loop(0, rpt)
        def _(i):
            stage[i, :] = zero

        pltpu.sync_copy(stage, acc.at[pl.ds(s * rpt, rpt)])
        plsc.subcore_barrier()

        gds = [None, None]
        sds = [None, None]

        def stage_in(slot, t):
            off = chunk0 + t * _QS
            pltpu.sync_copy(meta_hbm.at[pl.ds(off, _QS)], metab.at[slot])
            gds[slot] = [
                pltpu.async_copy(h_hbm.at[metab.at[slot, q, 0]],
                                 rowsb.at[slot, q], gsem[slot])
                for q in range(_QS)
            ]

        def scale(slot):
            gpc = _CHUNK // _L

            @pl.loop(0, _QS * gpc)
            def _(g):
                q = g // gpc
                r = g - q * gpc
                wvec = plsc.bitcast(
                    metab[slot, q, 2, pl.ds(r * _L, _L)], jnp.float32)
                for i in range(_L):
                    wv = wvec[i]
                    eidx = r * _L + i
                    rowsb[slot, q, eidx, :] = rowsb[slot, q, eidx, :] * wv

        def fire_scatter(slot):
            sds[slot] = [
                pltpu.async_copy(rowsb.at[slot, q],
                                 acc.at[metab.at[slot, q, 1]],
                                 ssem[slot], add=True)
                for q in range(_QS)
            ]

        stage_in(0, 0)
        for t in range(nsuper):
            slot = t & 1
            if t + 1 < nsuper:
                if sds[slot ^ 1] is not None:
                    for d in sds[slot ^ 1]:
                        d.wait()
                    sds[slot ^ 1] = None
                stage_in(slot ^ 1, t + 1)
            for d in gds[slot]:
                d.wait()
            gds[slot] = None
            scale(slot)
            fire_scatter(slot)
        for slot in (0, 1):
            if sds[slot] is not None:
                for d in sds[slot]:
                    d.wait()
                sds[slot] = None

        plsc.subcore_barrier()
        pltpu.sync_copy(acc.at[pl.ds(s * rpt, rpt)], stage)
        pltpu.sync_copy(stage, out_hbm.at[c, pl.ds(s * rpt, rpt)])

    return spmm(h, meta)


def _mm_body(a_ref, b_ref, o_ref):
    o_ref[...] = jnp.dot(a_ref[...], b_ref[...],
                         preferred_element_type=jnp.float32)


def _dense_mm(a, b):
    return pl.pallas_call(
        _mm_body,
        out_shape=jax.ShapeDtypeStruct((a.shape[0], b.shape[1]), jnp.float32),
    )(a, b)


def _layer2_body(p_ref, b1_ref, w2_ref, o_ref):
    hid = jnp.maximum(p_ref[0] + p_ref[1] + b1_ref[...], 0.0)
    o_ref[...] = jnp.dot(hid, w2_ref[...], preferred_element_type=jnp.float32)


def _layer2(partials, b1row, w2pad):
    npad = partials.shape[1]
    return pl.pallas_call(
        _layer2_body,
        out_shape=jax.ShapeDtypeStruct((npad, _L), jnp.float32),
    )(partials, b1row, w2pad)


def _final_body(p_ref, b2_ref, o_ref):
    o_ref[...] = p_ref[0] + p_ref[1] + b2_ref[...]


def _final(partials, b2row):
    npad = partials.shape[1]
    return pl.pallas_call(
        _final_body,
        out_shape=jax.ShapeDtypeStruct((npad, _L), jnp.float32),
    )(partials, b2row)


def kernel(x, edge_index, edge_weight, W1, b1, W2, b2):
    n, d = x.shape
    h1w = W1.shape[1]
    h2w = W2.shape[1]
    e = edge_index.shape[1]

    npad = _round_up(n, _NS * 8)

    per_w = _round_up(_round_up(e, _NW) // _NW, _SUPER)
    epad = _NW * per_w
    pad = epad - e
    src2d = jnp.pad(edge_index[0].astype(jnp.int32), (0, pad)).reshape(
        -1, _CHUNK)
    dst2d = jnp.pad(edge_index[1].astype(jnp.int32), (0, pad)).reshape(
        -1, _CHUNK)
    w2d = lax.bitcast_convert_type(
        jnp.pad(edge_weight.astype(jnp.float32), (0, pad)), jnp.int32
    ).reshape(-1, _CHUNK)
    meta = jnp.stack([src2d, dst2d, w2d], axis=1)

    w1pad = jnp.pad(W1, ((0, 0), (0, _L - h1w)))
    b1row = jnp.pad(b1, (0, _L - h1w)).reshape(1, _L)
    w2pad = jnp.pad(W2, ((0, _L - h1w), (0, _L - h2w)))
    b2row = jnp.pad(b2, (0, _L - h2w)).reshape(1, _L)

    s1 = _dense_mm(x, w1pad)
    p1 = _spmm_sc(s1, meta, npad)
    s2 = _layer2(p1, b1row, w2pad)
    p2 = _spmm_sc(s2, meta, npad)
    outp = _final(p2, b2row)
    return outp[:n, :h2w]


# speedup vs baseline: 2.0780x; 1.4805x over previous
import functools

import jax
import jax.numpy as jnp
from jax import lax
from jax.experimental import pallas as pl
from jax.experimental.pallas import tpu as pltpu
from jax.experimental.pallas import tpu_sc as plsc

_NC = 2
_NS = 16
_NW = _NC * _NS
_L = 16

_CHUNK = 128
_QS = 8
_SUPER = _CHUNK * _QS


def _round_up(v, m):
    return (v + m - 1) // m * m


def _spmm_sc(h, meta, npad):
    nchunks = meta.shape[0]
    per_w_chunks = nchunks // _NW
    nsuper = per_w_chunks // _QS
    rpt = npad // _NS

    mesh = plsc.VectorSubcoreMesh(core_axis_name="c", subcore_axis_name="s")

    @functools.partial(
        pl.kernel,
        out_type=jax.ShapeDtypeStruct((_NC, npad, _L), jnp.float32),
        mesh=mesh,
        scratch_types=[
            pltpu.VMEM((2, _QS, 3, _CHUNK), jnp.int32),
            pltpu.VMEM((2, _QS, _CHUNK, _L), jnp.float32),
            pltpu.VMEM((rpt, _L), jnp.float32),
            pltpu.VMEM_SHARED((npad, _L), jnp.float32),
            pltpu.SemaphoreType.DMA,
            pltpu.SemaphoreType.DMA,
            pltpu.SemaphoreType.DMA,
            pltpu.SemaphoreType.DMA,
        ],
        compiler_params=pltpu.CompilerParams(
            use_tc_tiling_on_sc=False, needs_layout_passes=False),
    )
    def spmm(h_hbm, meta_hbm, out_hbm,
             metab, rowsb, stage, acc, g0, g1, s0, s1):
        c = lax.axis_index("c")
        s = lax.axis_index("s")
        wid = c * _NS + s
        chunk0 = wid * per_w_chunks
        gsem = (g0, g1)
        ssem = (s0, s1)

        zero = jnp.zeros((_L,), jnp.float32)

        @pl.loop(0, rpt)
        def _(i):
            stage[i, :] = zero

        pltpu.sync_copy(stage, acc.at[pl.ds(s * rpt, rpt)])
        plsc.subcore_barrier()

        gds = [None, None]
        sds = [None, None]

        def stage_in(slot, t):
            off = chunk0 + t * _QS
            pltpu.sync_copy(meta_hbm.at[pl.ds(off, _QS)], metab.at[slot])
            gds[slot] = [
                pltpu.async_copy(h_hbm.at[metab.at[slot, q, 0]],
                                 rowsb.at[slot, q], gsem[slot])
                for q in range(_QS)
            ]

        def scale(slot):
            gpc = _CHUNK // _L

            @pl.loop(0, _QS * gpc)
            def _(g):
                q = g // gpc
                r = g - q * gpc
                wvec = plsc.bitcast(
                    metab[slot, q, 2, pl.ds(r * _L, _L)], jnp.float32)
                for i in range(_L):
                    wv = wvec[i]
                    eidx = r * _L + i
                    rowsb[slot, q, eidx, :] = rowsb[slot, q, eidx, :] * wv

        def fire_scatter(slot):
            sds[slot] = [
                pltpu.async_copy(rowsb.at[slot, q],
                                 acc.at[metab.at[slot, q, 1]],
                                 ssem[slot], add=True)
                for q in range(_QS)
            ]

        stage_in(0, 0)
        for t in range(nsuper):
            slot = t & 1
            if t + 1 < nsuper:
                if sds[slot ^ 1] is not None:
                    for d in sds[slot ^ 1]:
                        d.wait()
                    sds[slot ^ 1] = None
                stage_in(slot ^ 1, t + 1)
            for d in gds[slot]:
                d.wait()
            gds[slot] = None
            scale(slot)
            fire_scatter(slot)
        for slot in (0, 1):
            if sds[slot] is not None:
                for d in sds[slot]:
                    d.wait()
                sds[slot] = None

        plsc.subcore_barrier()
        pltpu.sync_copy(acc.at[pl.ds(s * rpt, rpt)], stage)
        pltpu.sync_copy(stage, out_hbm.at[c, pl.ds(s * rpt, rpt)])

    return spmm(h, meta)


def _mm_body(a_ref, b_ref, o_ref):
    o_ref[...] = jnp.dot(a_ref[...], b_ref[...],
                         preferred_element_type=jnp.float32)


def _dense_mm(a, b):
    return pl.pallas_call(
        _mm_body,
        out_shape=jax.ShapeDtypeStruct((a.shape[0], b.shape[1]), jnp.float32),
    )(a, b)


def _layer2_body(p_ref, b1_ref, w2_ref, o_ref):
    hid = jnp.maximum(p_ref[0] + p_ref[1] + b1_ref[...], 0.0)
    o_ref[...] = jnp.dot(hid, w2_ref[...], preferred_element_type=jnp.float32)


def _layer2(partials, b1row, w2pad):
    npad = partials.shape[1]
    return pl.pallas_call(
        _layer2_body,
        out_shape=jax.ShapeDtypeStruct((npad, _L), jnp.float32),
    )(partials, b1row, w2pad)


def _final_body(p_ref, b2_ref, o_ref):
    o_ref[...] = p_ref[0] + p_ref[1] + b2_ref[...]


def _final(partials, b2row):
    npad = partials.shape[1]
    return pl.pallas_call(
        _final_body,
        out_shape=jax.ShapeDtypeStruct((npad, _L), jnp.float32),
    )(partials, b2row)


def kernel(x, edge_index, edge_weight, W1, b1, W2, b2):
    n, d = x.shape
    h1w = W1.shape[1]
    h2w = W2.shape[1]
    e = edge_index.shape[1]

    npad = _round_up(n, _NS * 8)

    per_w = _round_up(_round_up(e, _NW) // _NW, _SUPER)
    epad = _NW * per_w
    pad = epad - e
    fill = jnp.arange(pad, dtype=jnp.int32)
    src2d = jnp.concatenate(
        [edge_index[0].astype(jnp.int32), fill % n]).reshape(-1, _CHUNK)
    dst2d = jnp.concatenate(
        [edge_index[1].astype(jnp.int32), n + fill % (npad - n)]).reshape(
            -1, _CHUNK)
    w2d = lax.bitcast_convert_type(
        jnp.pad(edge_weight.astype(jnp.float32), (0, pad)), jnp.int32
    ).reshape(-1, _CHUNK)
    meta = jnp.stack([src2d, dst2d, w2d], axis=1)

    w1pad = jnp.pad(W1, ((0, 0), (0, _L - h1w)))
    b1row = jnp.pad(b1, (0, _L - h1w)).reshape(1, _L)
    w2pad = jnp.pad(W2, ((0, _L - h1w), (0, _L - h2w)))
    b2row = jnp.pad(b2, (0, _L - h2w)).reshape(1, _L)

    s1 = _dense_mm(x, w1pad)
    p1 = _spmm_sc(s1, meta, npad)
    s2 = _layer2(p1, b1row, w2pad)
    p2 = _spmm_sc(s2, meta, npad)
    outp = _final(p2, b2row)
    return outp[:n, :h2w]


# speedup vs baseline: 2.1520x; 1.0356x over previous
import functools

import jax
import jax.numpy as jnp
from jax import lax
from jax.experimental import pallas as pl
from jax.experimental.pallas import tpu as pltpu
from jax.experimental.pallas import tpu_sc as plsc

_NC = 2
_NS = 16
_NW = _NC * _NS
_L = 16

_CHUNK = 128
_QS = 8
_SUPER = _CHUNK * _QS


def _round_up(v, m):
    return (v + m - 1) // m * m


def _spmm_sc(h, src1, dst1, w1, npad):
    epad = src1.shape[0]
    per_w = epad // _NW
    nsuper = per_w // _SUPER
    rpt = npad // _NS

    mesh = plsc.VectorSubcoreMesh(core_axis_name="c", subcore_axis_name="s")

    @functools.partial(
        pl.kernel,
        out_type=jax.ShapeDtypeStruct((_NC, npad, _L), jnp.float32),
        mesh=mesh,
        scratch_types=[
            pltpu.VMEM((2, _SUPER), jnp.int32),
            pltpu.VMEM((2, _QS, _CHUNK), jnp.int32),
            pltpu.VMEM((2, _SUPER), jnp.float32),
            pltpu.VMEM((2, _QS, _CHUNK, _L), jnp.float32),
            pltpu.VMEM((rpt, _L), jnp.float32),
            pltpu.VMEM_SHARED((npad, _L), jnp.float32),
            pltpu.SemaphoreType.DMA,
            pltpu.SemaphoreType.DMA,
            pltpu.SemaphoreType.DMA,
            pltpu.SemaphoreType.DMA,
            pltpu.SemaphoreType.DMA,
            pltpu.SemaphoreType.DMA,
        ],
        compiler_params=pltpu.CompilerParams(use_tc_tiling_on_sc=False),
    )
    def spmm(h_hbm, src_hbm, dst_hbm, w_hbm, out_hbm,
             srcb, dstb, wb, rowsb, stage, acc, m0, m1, g0, g1, s0, s1):
        c = lax.axis_index("c")
        s = lax.axis_index("s")
        wid = c * _NS + s
        ebase = wid * per_w
        msem = (m0, m1)
        gsem = (g0, g1)
        ssem = (s0, s1)

        zero = jnp.zeros((_L,), jnp.float32)

        @pl.loop(0, rpt)
        def _(i):
            stage[i, :] = zero

        pltpu.sync_copy(stage, acc.at[pl.ds(s * rpt, rpt)])
        plsc.subcore_barrier()

        mds = [None, None]
        gds = [None, None]
        sds = [None, None]

        def stage_meta(slot, t):
            off = ebase + t * _SUPER
            mds[slot] = [
                pltpu.async_copy(src_hbm.at[pl.ds(off, _SUPER)],
                                 srcb.at[slot], msem[slot]),
                pltpu.async_copy(w_hbm.at[pl.ds(off, _SUPER)],
                                 wb.at[slot], msem[slot]),
            ] + [
                pltpu.async_copy(dst_hbm.at[pl.ds(off + q * _CHUNK, _CHUNK)],
                                 dstb.at[slot, q], msem[slot])
                for q in range(_QS)
            ]

        def fire_gather(slot):
            for d in mds[slot]:
                d.wait()
            mds[slot] = None
            gds[slot] = [
                pltpu.async_copy(
                    h_hbm.at[srcb.at[slot, pl.ds(q * _CHUNK, _CHUNK)]],
                    rowsb.at[slot, q], gsem[slot])
                for q in range(_QS)
            ]

        def scale(slot):
            gpc = _CHUNK // _L

            @pl.loop(0, _QS * gpc)
            def _(g):
                q = g // gpc
                r = g - q * gpc
                wvec = wb[slot, pl.ds(g * _L, _L)]
                for i in range(_L):
                    wv = wvec[i]
                    eidx = r * _L + i
                    rowsb[slot, q, eidx, :] = rowsb[slot, q, eidx, :] * wv

        def fire_scatter(slot):
            sds[slot] = [
                pltpu.async_copy(rowsb.at[slot, q],
                                 acc.at[dstb.at[slot, q]],
                                 ssem[slot], add=True)
                for q in range(_QS)
            ]

        stage_meta(0, 0)
        fire_gather(0)
        for t in range(nsuper):
            slot = t & 1
            if t + 1 < nsuper:
                if sds[slot ^ 1] is not None:
                    for d in sds[slot ^ 1]:
                        d.wait()
                    sds[slot ^ 1] = None
                stage_meta(slot ^ 1, t + 1)
                fire_gather(slot ^ 1)
            for d in gds[slot]:
                d.wait()
            gds[slot] = None
            scale(slot)
            fire_scatter(slot)
        for slot in (0, 1):
            if sds[slot] is not None:
                for d in sds[slot]:
                    d.wait()
                sds[slot] = None

        plsc.subcore_barrier()
        pltpu.sync_copy(acc.at[pl.ds(s * rpt, rpt)], stage)
        pltpu.sync_copy(stage, out_hbm.at[c, pl.ds(s * rpt, rpt)])

    return spmm(h, src1, dst1, w1)


def _mm_body(a_ref, b_ref, o_ref):
    o_ref[...] = jnp.dot(a_ref[...], b_ref[...],
                         preferred_element_type=jnp.float32)


def _dense_mm(a, b):
    return pl.pallas_call(
        _mm_body,
        out_shape=jax.ShapeDtypeStruct((a.shape[0], b.shape[1]), jnp.float32),
    )(a, b)


def _layer2_body(p_ref, b1_ref, w2_ref, o_ref):
    hid = jnp.maximum(p_ref[0] + p_ref[1] + b1_ref[...], 0.0)
    o_ref[...] = jnp.dot(hid, w2_ref[...], preferred_element_type=jnp.float32)


def _layer2(partials, b1row, w2pad):
    npad = partials.shape[1]
    return pl.pallas_call(
        _layer2_body,
        out_shape=jax.ShapeDtypeStruct((npad, _L), jnp.float32),
    )(partials, b1row, w2pad)


def _final_body(p_ref, b2_ref, o_ref):
    o_ref[...] = p_ref[0] + p_ref[1] + b2_ref[...]


def _final(partials, b2row):
    npad = partials.shape[1]
    return pl.pallas_call(
        _final_body,
        out_shape=jax.ShapeDtypeStruct((npad, _L), jnp.float32),
    )(partials, b2row)


def kernel(x, edge_index, edge_weight, W1, b1, W2, b2):
    n, d = x.shape
    h1w = W1.shape[1]
    h2w = W2.shape[1]
    e = edge_index.shape[1]

    npad = _round_up(n, _NS * 8)

    per_w = _round_up(_round_up(e, _NW) // _NW, _SUPER)
    epad = _NW * per_w
    pad = epad - e
    fill = jnp.arange(pad, dtype=jnp.int32)
    src1 = jnp.concatenate([edge_index[0].astype(jnp.int32), fill % n])
    dst1 = jnp.concatenate(
        [edge_index[1].astype(jnp.int32), n + fill % (npad - n)])
    w1 = jnp.pad(edge_weight.astype(jnp.float32), (0, pad))

    w1pad = jnp.pad(W1, ((0, 0), (0, _L - h1w)))
    b1row = jnp.pad(b1, (0, _L - h1w)).reshape(1, _L)
    w2pad = jnp.pad(W2, ((0, _L - h1w), (0, _L - h2w)))
    b2row = jnp.pad(b2, (0, _L - h2w)).reshape(1, _L)

    s1 = _dense_mm(x, w1pad)
    p1 = _spmm_sc(s1, src1, dst1, w1, npad)
    s2 = _layer2(p1, b1row, w2pad)
    p2 = _spmm_sc(s2, src1, dst1, w1, npad)
    outp = _final(p2, b2row)
    return outp[:n, :h2w]


# speedup vs baseline: 2.4951x; 1.1594x over previous
import functools

import jax
import jax.numpy as jnp
from jax import lax
from jax.experimental import pallas as pl
from jax.experimental.pallas import tpu as pltpu
from jax.experimental.pallas import tpu_sc as plsc

_NC = 2
_NS = 16
_NW = _NC * _NS
_L = 16

_CHUNK = 128
_QS = 8
_SUPER = _CHUNK * _QS


def _round_up(v, m):
    return (v + m - 1) // m * m


def _spmm_sc(h, src1, dst1, w1, npad):
    epad = src1.shape[0]
    per_w = epad // _NW
    nsuper = per_w // _SUPER
    rpt = npad // _NS

    mesh = plsc.VectorSubcoreMesh(core_axis_name="c", subcore_axis_name="s")

    @functools.partial(
        pl.kernel,
        out_type=jax.ShapeDtypeStruct((_NC, npad // 8, 128), jnp.float32),
        mesh=mesh,
        scratch_types=[
            pltpu.VMEM((2, _SUPER), jnp.int32),
            pltpu.VMEM((2, _QS, _CHUNK), jnp.int32),
            pltpu.VMEM((2, _SUPER), jnp.float32),
            pltpu.VMEM((2, _QS, _CHUNK, _L), jnp.float32),
            pltpu.VMEM((rpt, _L), jnp.float32),
            pltpu.VMEM((rpt // 8, 128), jnp.float32),
            pltpu.VMEM_SHARED((npad, _L), jnp.float32),
            pltpu.SemaphoreType.DMA,
            pltpu.SemaphoreType.DMA,
            pltpu.SemaphoreType.DMA,
            pltpu.SemaphoreType.DMA,
            pltpu.SemaphoreType.DMA,
            pltpu.SemaphoreType.DMA,
        ],
        compiler_params=pltpu.CompilerParams(use_tc_tiling_on_sc=False),
    )
    def spmm(h_hbm, src_hbm, dst_hbm, w_hbm, out_hbm,
             srcb, dstb, wb, rowsb, stage, packed, acc,
             m0, m1, g0, g1, s0, s1):
        c = lax.axis_index("c")
        s = lax.axis_index("s")
        wid = c * _NS + s
        ebase = wid * per_w
        msem = (m0, m1)
        gsem = (g0, g1)
        ssem = (s0, s1)

        zero = jnp.zeros((_L,), jnp.float32)

        @pl.loop(0, rpt)
        def _(i):
            stage[i, :] = zero

        pltpu.sync_copy(stage, acc.at[pl.ds(s * rpt, rpt)])
        plsc.subcore_barrier()

        mds = [None, None]
        gds = [None, None]
        sds = [None, None]

        def stage_meta(slot, t):
            off = ebase + t * _SUPER
            mds[slot] = [
                pltpu.async_copy(src_hbm.at[pl.ds(off, _SUPER)],
                                 srcb.at[slot], msem[slot]),
                pltpu.async_copy(w_hbm.at[pl.ds(off, _SUPER)],
                                 wb.at[slot], msem[slot]),
            ] + [
                pltpu.async_copy(dst_hbm.at[pl.ds(off + q * _CHUNK, _CHUNK)],
                                 dstb.at[slot, q], msem[slot])
                for q in range(_QS)
            ]

        def fire_gather(slot):
            for d in mds[slot]:
                d.wait()
            mds[slot] = None
            gds[slot] = [
                pltpu.async_copy(
                    h_hbm.at[srcb.at[slot, pl.ds(q * _CHUNK, _CHUNK)]],
                    rowsb.at[slot, q], gsem[slot])
                for q in range(_QS)
            ]

        def scale(slot):
            gpc = _CHUNK // _L

            @pl.loop(0, _QS * gpc)
            def _(g):
                q = g // gpc
                r = g - q * gpc
                wvec = wb[slot, pl.ds(g * _L, _L)]
                for i in range(_L):
                    wv = wvec[i]
                    eidx = r * _L + i
                    rowsb[slot, q, eidx, :] = rowsb[slot, q, eidx, :] * wv

        def fire_scatter(slot):
            sds[slot] = [
                pltpu.async_copy(rowsb.at[slot, q],
                                 acc.at[dstb.at[slot, q]],
                                 ssem[slot], add=True)
                for q in range(_QS)
            ]

        stage_meta(0, 0)
        fire_gather(0)
        for t in range(nsuper):
            slot = t & 1
            if t + 1 < nsuper:
                if sds[slot ^ 1] is not None:
                    for d in sds[slot ^ 1]:
                        d.wait()
                    sds[slot ^ 1] = None
                stage_meta(slot ^ 1, t + 1)
                fire_gather(slot ^ 1)
            for d in gds[slot]:
                d.wait()
            gds[slot] = None
            scale(slot)
            fire_scatter(slot)
        for slot in (0, 1):
            if sds[slot] is not None:
                for d in sds[slot]:
                    d.wait()
                sds[slot] = None

        plsc.subcore_barrier()
        pltpu.sync_copy(acc.at[pl.ds(s * rpt, rpt)], stage)

        @pl.loop(0, rpt // 8)
        def _(i):
            for j in range(8):
                packed[i, pl.ds(j * _L, _L)] = stage[i * 8 + j, :]

        pltpu.sync_copy(packed,
                        out_hbm.at[c, pl.ds(s * (rpt // 8), rpt // 8)])

    return spmm(h, src1, dst1, w1)


def _mm_body(a_ref, b_ref, o_ref):
    o_ref[...] = jnp.dot(a_ref[...], b_ref[...],
                         preferred_element_type=jnp.float32)


def _dense_mm(a, b):
    return pl.pallas_call(
        _mm_body,
        out_shape=jax.ShapeDtypeStruct((a.shape[0], b.shape[1]), jnp.float32),
    )(a, b)


def _layer2_body(p_ref, b1_ref, w2_ref, o_ref):
    hid = jnp.maximum(p_ref[0] + p_ref[1] + b1_ref[...], 0.0)
    o_ref[...] = jnp.dot(hid, w2_ref[...], preferred_element_type=jnp.float32)


def _layer2(partials, b1t, w2bd):
    npr = partials.shape[1]
    return pl.pallas_call(
        _layer2_body,
        out_shape=jax.ShapeDtypeStruct((npr, 128), jnp.float32),
    )(partials, b1t, w2bd)


def _final_body(p_ref, b2_ref, o_ref):
    o_ref[...] = p_ref[0] + p_ref[1] + b2_ref[...]


def _final(partials, b2t):
    npr = partials.shape[1]
    return pl.pallas_call(
        _final_body,
        out_shape=jax.ShapeDtypeStruct((npr, 128), jnp.float32),
    )(partials, b2t)


def kernel(x, edge_index, edge_weight, W1, b1, W2, b2):
    n, d = x.shape
    h1w = W1.shape[1]
    h2w = W2.shape[1]
    e = edge_index.shape[1]

    npad = _round_up(n, _NS * 8)

    per_w = _round_up(_round_up(e, _NW) // _NW, _SUPER)
    epad = _NW * per_w
    pad = epad - e
    fill = jnp.arange(pad, dtype=jnp.int32)
    src1 = jnp.concatenate([edge_index[0].astype(jnp.int32), fill % n])
    dst1 = jnp.concatenate(
        [edge_index[1].astype(jnp.int32), n + fill % (npad - n)])
    w1 = jnp.pad(edge_weight.astype(jnp.float32), (0, pad))

    w1pad = jnp.pad(W1, ((0, 0), (0, _L - h1w)))
    b1t = jnp.tile(jnp.pad(b1, (0, _L - h1w)), 8).reshape(1, 128)
    w2bd = jnp.kron(jnp.eye(8, dtype=jnp.float32),
                    jnp.pad(W2, ((0, _L - h1w), (0, _L - h2w))))
    b2t = jnp.tile(jnp.pad(b2, (0, _L - h2w)), 8).reshape(1, 128)

    s1 = _dense_mm(x, w1pad)
    p1 = _spmm_sc(s1, src1, dst1, w1, npad)
    s2 = _layer2(p1, b1t, w2bd).reshape(npad, _L)
    p2 = _spmm_sc(s2, src1, dst1, w1, npad)
    outp = _final(p2, b2t)
    return outp.reshape(npad, _L)[:n, :h2w]


# speedup vs baseline: 2.5384x; 1.0174x over previous
import functools

import jax
import jax.numpy as jnp
from jax import lax
from jax.experimental import pallas as pl
from jax.experimental.pallas import tpu as pltpu
from jax.experimental.pallas import tpu_sc as plsc

_NC = 2
_NS = 16
_NW = _NC * _NS
_L = 16

_CHUNK = 128
_QS = 8
_SUPER = _CHUNK * _QS


def _round_up(v, m):
    return (v + m - 1) // m * m


def _spmm_sc(h, src1, dst1, w1, npad):
    epad = src1.shape[0]
    per_w = epad // _NW
    nfull = per_w // _SUPER
    rem = per_w - nfull * _SUPER
    tail_full = rem // _CHUNK
    tail_rem = rem % _CHUNK
    nsuper = nfull + (1 if rem else 0)
    rpt = npad // _NS

    mesh = plsc.VectorSubcoreMesh(core_axis_name="c", subcore_axis_name="s")

    @functools.partial(
        pl.kernel,
        out_type=jax.ShapeDtypeStruct((_NC, npad // 8, 128), jnp.float32),
        mesh=mesh,
        scratch_types=[
            pltpu.VMEM((2, _SUPER), jnp.int32),
            pltpu.VMEM((2, _QS, _CHUNK), jnp.int32),
            pltpu.VMEM((2, max(tail_rem, _L)), jnp.int32),
            pltpu.VMEM((2, _SUPER), jnp.float32),
            pltpu.VMEM((2, _QS, _CHUNK, _L), jnp.float32),
            pltpu.VMEM((rpt, _L), jnp.float32),
            pltpu.VMEM((rpt // 8, 128), jnp.float32),
            pltpu.VMEM_SHARED((npad, _L), jnp.float32),
            pltpu.SemaphoreType.DMA,
            pltpu.SemaphoreType.DMA,
            pltpu.SemaphoreType.DMA,
            pltpu.SemaphoreType.DMA,
            pltpu.SemaphoreType.DMA,
            pltpu.SemaphoreType.DMA,
        ],
        compiler_params=pltpu.CompilerParams(use_tc_tiling_on_sc=False),
    )
    def spmm(h_hbm, src_hbm, dst_hbm, w_hbm, out_hbm,
             srcb, dstb, dstbt, wb, rowsb, stage, packed, acc,
             m0, m1, g0, g1, s0, s1):
        c = lax.axis_index("c")
        s = lax.axis_index("s")
        wid = c * _NS + s
        ebase = wid * per_w
        msem = (m0, m1)
        gsem = (g0, g1)
        ssem = (s0, s1)

        zero = jnp.zeros((_L,), jnp.float32)

        @pl.loop(0, rpt)
        def _(i):
            stage[i, :] = zero

        pltpu.sync_copy(stage, acc.at[pl.ds(s * rpt, rpt)])
        plsc.subcore_barrier()

        mds = [None, None]
        gds = [None, None]
        sds = [None, None]

        def chunks_of(t):
            if t < nfull:
                return [(q, _CHUNK) for q in range(_QS)]
            cs = [(q, _CHUNK) for q in range(tail_full)]
            if tail_rem:
                cs.append((tail_full, tail_rem))
            return cs

        def nedges_of(t):
            return _SUPER if t < nfull else rem

        def stage_meta(slot, t):
            off = ebase + t * _SUPER
            ne = nedges_of(t)
            mds[slot] = [
                pltpu.async_copy(src_hbm.at[pl.ds(off, ne)],
                                 srcb.at[slot, pl.ds(0, ne)], msem[slot]),
                pltpu.async_copy(w_hbm.at[pl.ds(off, ne)],
                                 wb.at[slot, pl.ds(0, ne)], msem[slot]),
            ]
            for q, sz in chunks_of(t):
                dref = dstb.at[slot, q] if sz == _CHUNK else dstbt.at[slot]
                mds[slot].append(
                    pltpu.async_copy(dst_hbm.at[pl.ds(off + q * _CHUNK, sz)],
                                     dref, msem[slot]))

        def fire_gather(slot, t):
            for d in mds[slot]:
                d.wait()
            mds[slot] = None
            gds[slot] = [
                pltpu.async_copy(
                    h_hbm.at[srcb.at[slot, pl.ds(q * _CHUNK, sz)]],
                    rowsb.at[slot, q] if sz == _CHUNK
                    else rowsb.at[slot, q, pl.ds(0, sz)],
                    gsem[slot])
                for q, sz in chunks_of(t)
            ]

        def scale(slot, t):
            gpc = _CHUNK // _L

            @pl.loop(0, nedges_of(t) // _L)
            def _(g):
                q = g // gpc
                r = g - q * gpc
                wvec = wb[slot, pl.ds(g * _L, _L)]
                for i in range(_L):
                    wv = wvec[i]
                    eidx = r * _L + i
                    rowsb[slot, q, eidx, :] = rowsb[slot, q, eidx, :] * wv

        def fire_scatter(slot, t):
            sds[slot] = [
                pltpu.async_copy(
                    rowsb.at[slot, q] if sz == _CHUNK
                    else rowsb.at[slot, q, pl.ds(0, sz)],
                    acc.at[dstb.at[slot, q] if sz == _CHUNK
                           else dstbt.at[slot]],
                    ssem[slot], add=True)
                for q, sz in chunks_of(t)
            ]

        stage_meta(0, 0)
        fire_gather(0, 0)
        for t in range(nsuper):
            slot = t & 1
            if t + 1 < nsuper:
                if sds[slot ^ 1] is not None:
                    for d in sds[slot ^ 1]:
                        d.wait()
                    sds[slot ^ 1] = None
                stage_meta(slot ^ 1, t + 1)
                fire_gather(slot ^ 1, t + 1)
            for d in gds[slot]:
                d.wait()
            gds[slot] = None
            scale(slot, t)
            fire_scatter(slot, t)
        for slot in (0, 1):
            if sds[slot] is not None:
                for d in sds[slot]:
                    d.wait()
                sds[slot] = None

        plsc.subcore_barrier()
        pltpu.sync_copy(acc.at[pl.ds(s * rpt, rpt)], stage)

        @pl.loop(0, rpt // 8)
        def _(i):
            for j in range(8):
                packed[i, pl.ds(j * _L, _L)] = stage[i * 8 + j, :]

        pltpu.sync_copy(packed,
                        out_hbm.at[c, pl.ds(s * (rpt // 8), rpt // 8)])

    return spmm(h, src1, dst1, w1)


def _mm_body(a_ref, b_ref, o_ref):
    o_ref[...] = jnp.dot(a_ref[...], b_ref[...],
                         preferred_element_type=jnp.float32)


def _dense_mm(a, b):
    return pl.pallas_call(
        _mm_body,
        out_shape=jax.ShapeDtypeStruct((a.shape[0], b.shape[1]), jnp.float32),
    )(a, b)


def _layer2_body(p_ref, b1_ref, w2_ref, o_ref):
    hid = jnp.maximum(p_ref[0] + p_ref[1] + b1_ref[...], 0.0)
    o_ref[...] = jnp.dot(hid, w2_ref[...], preferred_element_type=jnp.float32)


def _layer2(partials, b1t, w2bd):
    npr = partials.shape[1]
    return pl.pallas_call(
        _layer2_body,
        out_shape=jax.ShapeDtypeStruct((npr, 128), jnp.float32),
    )(partials, b1t, w2bd)


def _final_body(p_ref, b2_ref, o_ref):
    o_ref[...] = p_ref[0] + p_ref[1] + b2_ref[...]


def _final(partials, b2t):
    npr = partials.shape[1]
    return pl.pallas_call(
        _final_body,
        out_shape=jax.ShapeDtypeStruct((npr, 128), jnp.float32),
    )(partials, b2t)


def kernel(x, edge_index, edge_weight, W1, b1, W2, b2):
    n, d = x.shape
    h1w = W1.shape[1]
    h2w = W2.shape[1]
    e = edge_index.shape[1]

    npad = _round_up(n, _NS * 8)

    src1 = edge_index[0].astype(jnp.int32)
    dst1 = edge_index[1].astype(jnp.int32)
    w1 = edge_weight.astype(jnp.float32)
    egrain = _NW * _L
    if e % egrain:
        pad = egrain - e % egrain
        fill = jnp.arange(pad, dtype=jnp.int32)
        src1 = jnp.concatenate([src1, fill % n])
        dst1 = jnp.concatenate([dst1, n + fill % (npad - n)])
        w1 = jnp.pad(w1, (0, pad))

    w1pad = jnp.pad(W1, ((0, 0), (0, _L - h1w)))
    b1t = jnp.tile(jnp.pad(b1, (0, _L - h1w)), 8).reshape(1, 128)
    w2bd = jnp.kron(jnp.eye(8, dtype=jnp.float32),
                    jnp.pad(W2, ((0, _L - h1w), (0, _L - h2w))))
    b2t = jnp.tile(jnp.pad(b2, (0, _L - h2w)), 8).reshape(1, 128)

    s1 = _dense_mm(x, w1pad)
    p1 = _spmm_sc(s1, src1, dst1, w1, npad)
    s2 = _layer2(p1, b1t, w2bd).reshape(npad, _L)
    p2 = _spmm_sc(s2, src1, dst1, w1, npad)
    outp = _final(p2, b2t)
    return outp.reshape(npad, _L)[:n, :h2w]


# speedup vs baseline: 2.7532x; 1.0846x over previous
import functools

import jax
import jax.numpy as jnp
from jax import lax
from jax.experimental import pallas as pl
from jax.experimental.pallas import tpu as pltpu
from jax.experimental.pallas import tpu_sc as plsc

_NC = 2
_NS = 16
_NW = _NC * _NS
_L = 16

_CHUNK = 128
_QS = 8
_SUPER = _CHUNK * _QS


def _round_up(v, m):
    return (v + m - 1) // m * m


def _spmm_sc(h, ei, w1, npad):
    epad = w1.shape[0]
    per_w = epad // _NW
    nfull = per_w // _SUPER
    rem = per_w - nfull * _SUPER
    tail_full = rem // _CHUNK
    tail_rem = rem % _CHUNK
    nsuper = nfull + (1 if rem else 0)
    rpt = npad // _NS

    mesh = plsc.VectorSubcoreMesh(core_axis_name="c", subcore_axis_name="s")

    @functools.partial(
        pl.kernel,
        out_type=jax.ShapeDtypeStruct((_NC, npad // 8, 128), jnp.float32),
        mesh=mesh,
        scratch_types=[
            pltpu.VMEM((2, _SUPER), jnp.int32),
            pltpu.VMEM((2, _QS, _CHUNK), jnp.int32),
            pltpu.VMEM((2, max(tail_rem, _L)), jnp.int32),
            pltpu.VMEM((2, _SUPER), jnp.float32),
            pltpu.VMEM((2, _QS, _CHUNK, _L), jnp.float32),
            pltpu.VMEM((rpt, _L), jnp.float32),
            pltpu.VMEM((rpt // 8, 128), jnp.float32),
            pltpu.VMEM_SHARED((npad, _L), jnp.float32),
            pltpu.SemaphoreType.DMA,
            pltpu.SemaphoreType.DMA,
            pltpu.SemaphoreType.DMA,
            pltpu.SemaphoreType.DMA,
            pltpu.SemaphoreType.DMA,
            pltpu.SemaphoreType.DMA,
        ],
        compiler_params=pltpu.CompilerParams(use_tc_tiling_on_sc=False),
    )
    def spmm(h_hbm, ei_hbm, w_hbm, out_hbm,
             srcb, dstb, dstbt, wb, rowsb, stage, packed, acc,
             m0, m1, g0, g1, s0, s1):
        c = lax.axis_index("c")
        s = lax.axis_index("s")
        wid = c * _NS + s
        ebase = wid * per_w
        msem = (m0, m1)
        gsem = (g0, g1)
        ssem = (s0, s1)

        zero = jnp.zeros((_L,), jnp.float32)

        @pl.loop(0, rpt)
        def _(i):
            stage[i, :] = zero

        pltpu.sync_copy(stage, acc.at[pl.ds(s * rpt, rpt)])
        plsc.subcore_barrier()

        mds = [None, None]
        gds = [None, None]
        sds = [None, None]

        def chunks_of(t):
            if t < nfull:
                return [(q, _CHUNK) for q in range(_QS)]
            cs = [(q, _CHUNK) for q in range(tail_full)]
            if tail_rem:
                cs.append((tail_full, tail_rem))
            return cs

        def nedges_of(t):
            return _SUPER if t < nfull else rem

        def stage_meta(slot, t):
            off = ebase + t * _SUPER
            ne = nedges_of(t)
            mds[slot] = [
                pltpu.async_copy(ei_hbm.at[pl.ds(off, ne)],
                                 srcb.at[slot, pl.ds(0, ne)], msem[slot]),
                pltpu.async_copy(w_hbm.at[pl.ds(off, ne)],
                                 wb.at[slot, pl.ds(0, ne)], msem[slot]),
            ]
            for q, sz in chunks_of(t):
                dref = dstb.at[slot, q] if sz == _CHUNK else dstbt.at[slot]
                mds[slot].append(
                    pltpu.async_copy(
                        ei_hbm.at[pl.ds(epad + off + q * _CHUNK, sz)],
                        dref, msem[slot]))

        def fire_gather(slot, t):
            for d in mds[slot]:
                d.wait()
            mds[slot] = None
            gds[slot] = [
                pltpu.async_copy(
                    h_hbm.at[srcb.at[slot, pl.ds(q * _CHUNK, sz)]],
                    rowsb.at[slot, q] if sz == _CHUNK
                    else rowsb.at[slot, q, pl.ds(0, sz)],
                    gsem[slot])
                for q, sz in chunks_of(t)
            ]

        def scale(slot, t):
            gpc = _CHUNK // _L

            @pl.loop(0, nedges_of(t) // _L)
            def _(g):
                q = g // gpc
                r = g - q * gpc
                wvec = wb[slot, pl.ds(g * _L, _L)]
                for i in range(_L):
                    wv = wvec[i]
                    eidx = r * _L + i
                    rowsb[slot, q, eidx, :] = rowsb[slot, q, eidx, :] * wv

        def fire_scatter(slot, t):
            sds[slot] = [
                pltpu.async_copy(
                    rowsb.at[slot, q] if sz == _CHUNK
                    else rowsb.at[slot, q, pl.ds(0, sz)],
                    acc.at[dstb.at[slot, q] if sz == _CHUNK
                           else dstbt.at[slot]],
                    ssem[slot], add=True)
                for q, sz in chunks_of(t)
            ]

        stage_meta(0, 0)
        fire_gather(0, 0)
        for t in range(nsuper):
            slot = t & 1
            if t + 1 < nsuper:
                if sds[slot ^ 1] is not None:
                    for d in sds[slot ^ 1]:
                        d.wait()
                    sds[slot ^ 1] = None
                stage_meta(slot ^ 1, t + 1)
                fire_gather(slot ^ 1, t + 1)
            for d in gds[slot]:
                d.wait()
            gds[slot] = None
            scale(slot, t)
            fire_scatter(slot, t)
        for slot in (0, 1):
            if sds[slot] is not None:
                for d in sds[slot]:
                    d.wait()
                sds[slot] = None

        plsc.subcore_barrier()
        pltpu.sync_copy(acc.at[pl.ds(s * rpt, rpt)], stage)

        @pl.loop(0, rpt // 8)
        def _(i):
            for j in range(8):
                packed[i, pl.ds(j * _L, _L)] = stage[i * 8 + j, :]

        pltpu.sync_copy(packed,
                        out_hbm.at[c, pl.ds(s * (rpt // 8), rpt // 8)])

    return spmm(h, ei, w1)


def _mm_body(a_ref, b_ref, o_ref):
    o_ref[...] = jnp.dot(a_ref[...], b_ref[...],
                         preferred_element_type=jnp.float32)


def _dense_mm(a, b):
    return pl.pallas_call(
        _mm_body,
        out_shape=jax.ShapeDtypeStruct((a.shape[0], b.shape[1]), jnp.float32),
    )(a, b)


def _layer2_body(p_ref, b1_ref, w2_ref, o_ref):
    hid = jnp.maximum(p_ref[0] + p_ref[1] + b1_ref[...], 0.0)
    o_ref[...] = jnp.dot(hid, w2_ref[...], preferred_element_type=jnp.float32)


def _layer2(partials, b1t, w2bd):
    npr = partials.shape[1]
    return pl.pallas_call(
        _layer2_body,
        out_shape=jax.ShapeDtypeStruct((npr, 128), jnp.float32),
    )(partials, b1t, w2bd)


def _final_body(p_ref, b2_ref, o_ref):
    o_ref[...] = p_ref[0] + p_ref[1] + b2_ref[...]


def _final(partials, b2t):
    npr = partials.shape[1]
    return pl.pallas_call(
        _final_body,
        out_shape=jax.ShapeDtypeStruct((npr, 128), jnp.float32),
    )(partials, b2t)


def kernel(x, edge_index, edge_weight, W1, b1, W2, b2):
    n, d = x.shape
    h1w = W1.shape[1]
    h2w = W2.shape[1]
    e = edge_index.shape[1]

    npad = _round_up(n, _NS * 8)

    w1 = edge_weight.astype(jnp.float32)
    egrain = _NW * _L
    if e % egrain:
        pad = egrain - e % egrain
        fill = jnp.arange(pad, dtype=jnp.int32)
        ei = jnp.concatenate([
            edge_index[0].astype(jnp.int32), fill % n,
            edge_index[1].astype(jnp.int32), n + fill % (npad - n)])
        w1 = jnp.pad(w1, (0, pad))
    else:
        ei = edge_index.astype(jnp.int32).reshape(2 * e)

    w1pad = jnp.pad(W1, ((0, 0), (0, _L - h1w)))
    b1t = jnp.tile(jnp.pad(b1, (0, _L - h1w)), 8).reshape(1, 128)
    w2bd = jnp.kron(jnp.eye(8, dtype=jnp.float32),
                    jnp.pad(W2, ((0, _L - h1w), (0, _L - h2w))))
    b2t = jnp.tile(jnp.pad(b2, (0, _L - h2w)), 8).reshape(1, 128)

    s1 = _dense_mm(x, w1pad)
    p1 = _spmm_sc(s1, ei, w1, npad)
    s2 = _layer2(p1, b1t, w2bd).reshape(npad, _L)
    p2 = _spmm_sc(s2, ei, w1, npad)
    outp = _final(p2, b2t)
    return outp.reshape(npad, _L)[:n, :h2w]


# speedup vs baseline: 2.8270x; 1.0268x over previous
import functools

import jax
import jax.numpy as jnp
from jax import lax
from jax.experimental import pallas as pl
from jax.experimental.pallas import tpu as pltpu
from jax.experimental.pallas import tpu_sc as plsc

_NC = 2
_NS = 16
_NW = _NC * _NS
_L = 16

_CHUNK = 128
_QS = 16
_SUPER = _CHUNK * _QS


def _round_up(v, m):
    return (v + m - 1) // m * m


def _spmm_sc(h, ei, w1, npad):
    epad = w1.shape[0]
    per_w = epad // _NW
    nfull = per_w // _SUPER
    rem = per_w - nfull * _SUPER
    tail_full = rem // _CHUNK
    tail_rem = rem % _CHUNK
    nsuper = nfull + (1 if rem else 0)
    rpt = npad // _NS

    mesh = plsc.VectorSubcoreMesh(core_axis_name="c", subcore_axis_name="s")

    @functools.partial(
        pl.kernel,
        out_type=jax.ShapeDtypeStruct((_NC, npad // 8, 128), jnp.float32),
        mesh=mesh,
        scratch_types=[
            pltpu.VMEM((2, _SUPER), jnp.int32),
            pltpu.VMEM((2, _QS, _CHUNK), jnp.int32),
            pltpu.VMEM((2, max(tail_rem, _L)), jnp.int32),
            pltpu.VMEM((2, _SUPER), jnp.float32),
            pltpu.VMEM((2, _QS, _CHUNK, _L), jnp.float32),
            pltpu.VMEM((rpt, _L), jnp.float32),
            pltpu.VMEM((rpt // 8, 128), jnp.float32),
            pltpu.VMEM_SHARED((npad, _L), jnp.float32),
            pltpu.SemaphoreType.DMA,
            pltpu.SemaphoreType.DMA,
            pltpu.SemaphoreType.DMA,
            pltpu.SemaphoreType.DMA,
            pltpu.SemaphoreType.DMA,
            pltpu.SemaphoreType.DMA,
        ],
        compiler_params=pltpu.CompilerParams(use_tc_tiling_on_sc=False),
    )
    def spmm(h_hbm, ei_hbm, w_hbm, out_hbm,
             srcb, dstb, dstbt, wb, rowsb, stage, packed, acc,
             m0, m1, g0, g1, s0, s1):
        c = lax.axis_index("c")
        s = lax.axis_index("s")
        wid = c * _NS + s
        ebase = wid * per_w
        msem = (m0, m1)
        gsem = (g0, g1)
        ssem = (s0, s1)

        zero = jnp.zeros((_L,), jnp.float32)

        @pl.loop(0, rpt)
        def _(i):
            stage[i, :] = zero

        pltpu.sync_copy(stage, acc.at[pl.ds(s * rpt, rpt)])
        plsc.subcore_barrier()

        mds = [None, None]
        gds = [None, None]
        sds = [None, None]

        def chunks_of(t):
            if t < nfull:
                return [(q, _CHUNK) for q in range(_QS)]
            cs = [(q, _CHUNK) for q in range(tail_full)]
            if tail_rem:
                cs.append((tail_full, tail_rem))
            return cs

        def nedges_of(t):
            return _SUPER if t < nfull else rem

        def stage_meta(slot, t):
            off = ebase + t * _SUPER
            ne = nedges_of(t)
            mds[slot] = [
                pltpu.async_copy(ei_hbm.at[pl.ds(off, ne)],
                                 srcb.at[slot, pl.ds(0, ne)], msem[slot]),
                pltpu.async_copy(w_hbm.at[pl.ds(off, ne)],
                                 wb.at[slot, pl.ds(0, ne)], msem[slot]),
            ]
            for q, sz in chunks_of(t):
                dref = dstb.at[slot, q] if sz == _CHUNK else dstbt.at[slot]
                mds[slot].append(
                    pltpu.async_copy(
                        ei_hbm.at[pl.ds(epad + off + q * _CHUNK, sz)],
                        dref, msem[slot]))

        def fire_gather(slot, t):
            for d in mds[slot]:
                d.wait()
            mds[slot] = None
            gds[slot] = [
                pltpu.async_copy(
                    h_hbm.at[srcb.at[slot, pl.ds(q * _CHUNK, sz)]],
                    rowsb.at[slot, q] if sz == _CHUNK
                    else rowsb.at[slot, q, pl.ds(0, sz)],
                    gsem[slot])
                for q, sz in chunks_of(t)
            ]

        def scale(slot, t):
            gpc = _CHUNK // _L

            @pl.loop(0, nedges_of(t) // _L)
            def _(g):
                q = g // gpc
                r = g - q * gpc
                wvec = wb[slot, pl.ds(g * _L, _L)]
                for i in range(_L):
                    wv = wvec[i]
                    eidx = r * _L + i
                    rowsb[slot, q, eidx, :] = rowsb[slot, q, eidx, :] * wv

        def fire_scatter(slot, t):
            sds[slot] = [
                pltpu.async_copy(
                    rowsb.at[slot, q] if sz == _CHUNK
                    else rowsb.at[slot, q, pl.ds(0, sz)],
                    acc.at[dstb.at[slot, q] if sz == _CHUNK
                           else dstbt.at[slot]],
                    ssem[slot], add=True)
                for q, sz in chunks_of(t)
            ]

        stage_meta(0, 0)
        fire_gather(0, 0)
        for t in range(nsuper):
            slot = t & 1
            if t + 1 < nsuper:
                if sds[slot ^ 1] is not None:
                    for d in sds[slot ^ 1]:
                        d.wait()
                    sds[slot ^ 1] = None
                stage_meta(slot ^ 1, t + 1)
                fire_gather(slot ^ 1, t + 1)
            for d in gds[slot]:
                d.wait()
            gds[slot] = None
            scale(slot, t)
            fire_scatter(slot, t)
        for slot in (0, 1):
            if sds[slot] is not None:
                for d in sds[slot]:
                    d.wait()
                sds[slot] = None

        plsc.subcore_barrier()
        pltpu.sync_copy(acc.at[pl.ds(s * rpt, rpt)], stage)

        @pl.loop(0, rpt // 8)
        def _(i):
            for j in range(8):
                packed[i, pl.ds(j * _L, _L)] = stage[i * 8 + j, :]

        pltpu.sync_copy(packed,
                        out_hbm.at[c, pl.ds(s * (rpt // 8), rpt // 8)])

    return spmm(h, ei, w1)


def _mm_body(a_ref, b_ref, o_ref):
    o_ref[...] = jnp.dot(a_ref[...], b_ref[...],
                         preferred_element_type=jnp.float32)


def _dense_mm(a, b):
    return pl.pallas_call(
        _mm_body,
        out_shape=jax.ShapeDtypeStruct((a.shape[0], b.shape[1]), jnp.float32),
    )(a, b)


def _layer2_body(p_ref, b1_ref, w2_ref, o_ref):
    hid = jnp.maximum(p_ref[0] + p_ref[1] + b1_ref[...], 0.0)
    o_ref[...] = jnp.dot(hid, w2_ref[...], preferred_element_type=jnp.float32)


def _layer2(partials, b1t, w2bd):
    npr = partials.shape[1]
    return pl.pallas_call(
        _layer2_body,
        out_shape=jax.ShapeDtypeStruct((npr, 128), jnp.float32),
    )(partials, b1t, w2bd)


def _final_body(p_ref, b2_ref, o_ref):
    o_ref[...] = p_ref[0] + p_ref[1] + b2_ref[...]


def _final(partials, b2t):
    npr = partials.shape[1]
    return pl.pallas_call(
        _final_body,
        out_shape=jax.ShapeDtypeStruct((npr, 128), jnp.float32),
    )(partials, b2t)


def kernel(x, edge_index, edge_weight, W1, b1, W2, b2):
    n, d = x.shape
    h1w = W1.shape[1]
    h2w = W2.shape[1]
    e = edge_index.shape[1]

    npad = _round_up(n, _NS * 8)

    w1 = edge_weight.astype(jnp.float32)
    egrain = _NW * _L
    if e % egrain:
        pad = egrain - e % egrain
        fill = jnp.arange(pad, dtype=jnp.int32)
        ei = jnp.concatenate([
            edge_index[0].astype(jnp.int32), fill % n,
            edge_index[1].astype(jnp.int32), n + fill % (npad - n)])
        w1 = jnp.pad(w1, (0, pad))
    else:
        ei = edge_index.astype(jnp.int32).reshape(2 * e)

    w1pad = jnp.pad(W1, ((0, 0), (0, _L - h1w)))
    b1t = jnp.tile(jnp.pad(b1, (0, _L - h1w)), 8).reshape(1, 128)
    w2bd = jnp.kron(jnp.eye(8, dtype=jnp.float32),
                    jnp.pad(W2, ((0, _L - h1w), (0, _L - h2w))))
    b2t = jnp.tile(jnp.pad(b2, (0, _L - h2w)), 8).reshape(1, 128)

    s1 = _dense_mm(x, w1pad)
    p1 = _spmm_sc(s1, ei, w1, npad)
    s2 = _layer2(p1, b1t, w2bd).reshape(npad, _L)
    p2 = _spmm_sc(s2, ei, w1, npad)
    outp = _final(p2, b2t)
    return outp.reshape(npad, _L)[:n, :h2w]
